# baseline (speedup 1.0000x reference)
import functools

import jax
import jax.numpy as jnp
from jax import lax
from jax.experimental import pallas as pl
from jax.experimental.pallas import tpu as pltpu
from jax.experimental.pallas import tpu_sc as plsc

L = 16
NC = 2
NS = 16
CHUNK = 80


def _matmul_body(x_ref, w_ref, o_ref):
    o_ref[0] = jnp.dot(x_ref[...], w_ref[0], preferred_element_type=jnp.float32)


def _rel_transforms(x, weight):
    n, din = x.shape
    r, _, dout = weight.shape
    bn = 1000
    return pl.pallas_call(
        _matmul_body,
        grid=(r, n // bn),
        in_specs=[
            pl.BlockSpec((bn, din), lambda ri, ni: (ni, 0)),
            pl.BlockSpec((1, din, dout), lambda ri, ni: (ri, 0, 0)),
        ],
        out_specs=pl.BlockSpec((1, bn, dout), lambda ri, ni: (ri, ni, 0)),
        out_shape=jax.ShapeDtypeStruct((r, n, dout), jnp.float32),
    )(x, weight)


def _combine_body(p_ref, b_ref, o_ref):
    o_ref[...] = p_ref[0] + p_ref[1] + b_ref[...]


def _combine(partial, h_bias):
    nc, n, dout = partial.shape
    bn = 1000
    return pl.pallas_call(
        _combine_body,
        grid=(n // bn,),
        in_specs=[
            pl.BlockSpec((nc, bn, dout), lambda i: (0, i, 0)),
            pl.BlockSpec((1, dout), lambda i: (0, 0)),
        ],
        out_specs=pl.BlockSpec((bn, dout), lambda i: (i, 0)),
        out_shape=jax.ShapeDtypeStruct((n, dout), jnp.float32),
    )(partial, h_bias.reshape(1, dout))


def _make_sc_scatter(n, dout, e):
    nw = NC * NS
    ew = e // nw
    nchunks = ew // CHUNK
    rstride = 624
    rcnt = 640
    zrows = 128
    nzcopies = rcnt // zrows
    nj = dout // L
    mesh = plsc.VectorSubcoreMesh(core_axis_name="c", subcore_axis_name="s")

    mw = 3 * CHUNK
    gmax = nw * nchunks - 1

    @functools.partial(
        pl.kernel,
        out_type=jax.ShapeDtypeStruct((NC, n, dout), jnp.float32),
        mesh=mesh,
        scratch_types=[
            pltpu.VMEM((mw,), jnp.int32),
            pltpu.VMEM((mw,), jnp.int32),
            pltpu.VMEM((CHUNK,), jnp.int32),
            pltpu.VMEM((CHUNK,), jnp.int32),
            pltpu.VMEM((CHUNK,), jnp.int32),
            pltpu.VMEM((CHUNK,), jnp.int32),
            pltpu.VMEM((CHUNK,), jnp.float32),
            pltpu.VMEM((CHUNK,), jnp.float32),
            pltpu.VMEM((CHUNK, 128), jnp.float32),
            pltpu.VMEM((CHUNK, 128), jnp.float32),
            pltpu.VMEM_SHARED((n, dout), jnp.float32),
            pltpu.SemaphoreType.DMA,
            pltpu.SemaphoreType.DMA,
            pltpu.SemaphoreType.DMA,
            pltpu.SemaphoreType.DMA,
            pltpu.SemaphoreType.DMA,
            pltpu.SemaphoreType.DMA,
            pltpu.SemaphoreType.DMA,
            pltpu.SemaphoreType.DMA,
        ],
    )
    def sc_kernel(meta_hbm, norm_hbm, h_hbm, out_hbm,
                  meta_v0, meta_v1, idx_v0, idx_v1, dst_v0, dst_v1,
                  norm_v0, norm_v1, rows_v0, rows_v1,
                  acc, semm0, semm1, semn0, semn1, semg0, semg1,
                  sems0, sems1):
        meta_v = [meta_v0, meta_v1]
        idx_v = [idx_v0, idx_v1]
        dst_v = [dst_v0, dst_v1]
        norm_v = [norm_v0, norm_v1]
        rows_v = [rows_v0, rows_v1]
        semm = [semm0, semm1]
        semn = [semn0, semn1]
        semg = [semg0, semg1]
        sems = [sems0, sems1]
        c = lax.axis_index("c")
        s = lax.axis_index("s")
        wid = c * NS + s

        def meta_start(k, b):
            g = jnp.minimum(wid * nchunks + k, gmax)
            pltpu.async_copy(
                meta_hbm.at[pl.ds(g * mw, mw)], meta_v[b], semm[b])
            pltpu.async_copy(
                norm_hbm.at[pl.ds(g * CHUNK, CHUNK)], norm_v[b], semn[b])

        def meta_wait(b):
            pltpu.make_async_copy(
                meta_hbm.at[pl.ds(0, mw)], meta_v[b], semm[b]).wait()
            pltpu.make_async_copy(
                norm_hbm.at[pl.ds(0, CHUNK)], norm_v[b], semn[b]).wait()

        def idx_compute(b):
            for j in range(CHUNK // L):
                sl = pl.ds(j * L, L)
                idx_v[b][sl] = (meta_v[b][pl.ds(CHUNK + j * L, L)] * n
                                + meta_v[b][sl])

        def gather_start(b):
            return pltpu.async_copy(h_hbm.at[idx_v[b]], rows_v[b], semg[b])

        def gather_wait(b):
            pltpu.make_async_copy(
                h_hbm.at[idx_v[b]], rows_v[b], semg[b]).wait()

        def process_compute(b):
            for j in range(CHUNK // L):
                sl = pl.ds(j * L, L)
                dst_v[b][sl] = meta_v[b][pl.ds(2 * CHUNK + j * L, L)]

            def group_body(gi, _):
                nv = norm_v[b][pl.ds(gi * L, L)]
                for lane in range(L):
                    ei = gi * L + lane
                    nb = jnp.full((L,), nv[lane], jnp.float32)
                    for j in range(nj):
                        sl = pl.ds(j * L, L)
                        rows_v[b][ei, sl] = rows_v[b][ei, sl] * nb
                return 0
            lax.fori_loop(0, CHUNK // L, group_body, 0)

        def scatter_start(b):
            pltpu.async_copy(rows_v[b], acc.at[dst_v[b]], sems[b], add=True)

        def scatter_wait(b):
            pltpu.make_async_copy(rows_v[b], acc.at[dst_v[b]],
                                  sems[b]).wait()

        def zstore(i, _):
            for j in range(nj):
                rows_v0[i, pl.ds(j * L, L)] = jnp.zeros((L,), jnp.float32)
            return 0
        lax.fori_loop(0, CHUNK, zstore, 0)
        for t in range(rcnt // CHUNK):
            pltpu.sync_copy(rows_v0,
                            acc.at[pl.ds(s * rstride + t * CHUNK, CHUNK)])
        plsc.subcore_barrier()

        meta_start(0, 0)
        meta_wait(0)
        idx_compute(0)
        gather_start(0)
        meta_start(1, 1)

        def half_step(k, cur, nxt, first=False):
            meta_wait(nxt)
            idx_compute(nxt)
            gather_wait(cur)
            if not first:
                scatter_wait(nxt)
            gather_start(nxt)
            process_compute(cur)
            meta_start(k + 2, cur)
            scatter_start(cur)

        half_step(0, 0, 1, first=True)
        half_step(1, 1, 0)

        def pair_body(t, _):
            half_step(2 * t, 0, 1)
            half_step(2 * t + 1, 1, 0)
            return 0
        lax.fori_loop(1, (nchunks - 1) // 2, pair_body, 0)

        meta_wait(1)
        gather_wait(0)
        scatter_wait(1)
        process_compute(0)
        scatter_start(0)
        scatter_wait(0)

        plsc.subcore_barrier()
        pltpu.sync_copy(acc.at[pl.ds(s * rstride, rcnt)],
                        out_hbm.at[c, pl.ds(s * rstride, rcnt)])

    return sc_kernel


def kernel(g, x, etypes, norm, weight, h_bias):
    n, din = x.shape
    r, _, dout = weight.shape
    e = g.shape[1]
    nch = e // CHUNK
    meta = jnp.concatenate(
        [g[0].reshape(nch, CHUNK), etypes.reshape(nch, CHUNK),
         g[1].reshape(nch, CHUNK)],
        axis=1).reshape(-1)
    h_flat = _rel_transforms(x, weight).reshape(r * n, dout)
    partial = _make_sc_scatter(n, dout, e)(meta, norm.reshape(-1), h_flat)
    return _combine(partial, h_bias)


# speedup vs baseline: 1.1925x; 1.1925x over previous
import functools

import jax
import jax.numpy as jnp
from jax import lax
from jax.experimental import pallas as pl
from jax.experimental.pallas import tpu as pltpu
from jax.experimental.pallas import tpu_sc as plsc

L = 16
NC = 2
NS = 16
CHUNK = 80


def _make_matmul_body(r):
    def _matmul_body(x_ref, w_ref, o_ref):
        for ri in range(r):
            o_ref[ri] = jnp.dot(x_ref[...], w_ref[ri],
                                preferred_element_type=jnp.float32)
    return _matmul_body


def _rel_transforms(x, weight):
    n, din = x.shape
    r, _, dout = weight.shape
    bn = 1000
    return pl.pallas_call(
        _make_matmul_body(r),
        grid=(n // bn,),
        in_specs=[
            pl.BlockSpec((bn, din), lambda ni: (ni, 0)),
            pl.BlockSpec((r, din, dout), lambda ni: (0, 0, 0)),
        ],
        out_specs=pl.BlockSpec((r, bn, dout), lambda ni: (0, ni, 0)),
        out_shape=jax.ShapeDtypeStruct((r, n, dout), jnp.float32),
    )(x, weight)


def _combine_body(p_ref, b_ref, o_ref):
    o_ref[...] = p_ref[0] + p_ref[1] + b_ref[...]


def _combine(partial, h_bias):
    nc, n, dout = partial.shape
    bn = 1000
    return pl.pallas_call(
        _combine_body,
        grid=(n // bn,),
        in_specs=[
            pl.BlockSpec((nc, bn, dout), lambda i: (0, i, 0)),
            pl.BlockSpec((1, dout), lambda i: (0, 0)),
        ],
        out_specs=pl.BlockSpec((bn, dout), lambda i: (i, 0)),
        out_shape=jax.ShapeDtypeStruct((n, dout), jnp.float32),
    )(partial, h_bias.reshape(1, dout))


def _make_sc_scatter(n, dout, e):
    nw = NC * NS
    ew = e // nw
    nchunks = ew // CHUNK
    rstride = 624
    rcnt = 640
    zrows = 128
    nzcopies = rcnt // zrows
    nj = dout // L
    mesh = plsc.VectorSubcoreMesh(core_axis_name="c", subcore_axis_name="s")

    mw = 3 * CHUNK
    gmax = nw * nchunks - 1

    @functools.partial(
        pl.kernel,
        out_type=jax.ShapeDtypeStruct((NC, n, dout), jnp.float32),
        mesh=mesh,
        scratch_types=[
            pltpu.VMEM((mw,), jnp.int32),
            pltpu.VMEM((mw,), jnp.int32),
            pltpu.VMEM((CHUNK,), jnp.int32),
            pltpu.VMEM((CHUNK,), jnp.int32),
            pltpu.VMEM((CHUNK,), jnp.int32),
            pltpu.VMEM((CHUNK,), jnp.int32),
            pltpu.VMEM((CHUNK,), jnp.float32),
            pltpu.VMEM((CHUNK,), jnp.float32),
            pltpu.VMEM((CHUNK, 128), jnp.float32),
            pltpu.VMEM((CHUNK, 128), jnp.float32),
            pltpu.VMEM_SHARED((n, dout), jnp.float32),
            pltpu.SemaphoreType.DMA,
            pltpu.SemaphoreType.DMA,
            pltpu.SemaphoreType.DMA,
            pltpu.SemaphoreType.DMA,
            pltpu.SemaphoreType.DMA,
            pltpu.SemaphoreType.DMA,
            pltpu.SemaphoreType.DMA,
            pltpu.SemaphoreType.DMA,
        ],
    )
    def sc_kernel(meta_hbm, norm_hbm, h_hbm, out_hbm,
                  meta_v0, meta_v1, idx_v0, idx_v1, dst_v0, dst_v1,
                  norm_v0, norm_v1, rows_v0, rows_v1,
                  acc, semm0, semm1, semn0, semn1, semg0, semg1,
                  sems0, sems1):
        meta_v = [meta_v0, meta_v1]
        idx_v = [idx_v0, idx_v1]
        dst_v = [dst_v0, dst_v1]
        norm_v = [norm_v0, norm_v1]
        rows_v = [rows_v0, rows_v1]
        semm = [semm0, semm1]
        semn = [semn0, semn1]
        semg = [semg0, semg1]
        sems = [sems0, sems1]
        c = lax.axis_index("c")
        s = lax.axis_index("s")
        wid = c * NS + s

        def meta_start(k, b):
            g = jnp.minimum(wid * nchunks + k, gmax)
            pltpu.async_copy(
                meta_hbm.at[pl.ds(g * mw, mw)], meta_v[b], semm[b])
            pltpu.async_copy(
                norm_hbm.at[pl.ds(g * CHUNK, CHUNK)], norm_v[b], semn[b])

        def meta_wait(b):
            pltpu.make_async_copy(
                meta_hbm.at[pl.ds(0, mw)], meta_v[b], semm[b]).wait()
            pltpu.make_async_copy(
                norm_hbm.at[pl.ds(0, CHUNK)], norm_v[b], semn[b]).wait()

        def idx_compute(b):
            for j in range(CHUNK // L):
                sl = pl.ds(j * L, L)
                idx_v[b][sl] = (meta_v[b][pl.ds(CHUNK + j * L, L)] * n
                                + meta_v[b][sl])

        def gather_start(b):
            return pltpu.async_copy(h_hbm.at[idx_v[b]], rows_v[b], semg[b])

        def gather_wait(b):
            pltpu.make_async_copy(
                h_hbm.at[idx_v[b]], rows_v[b], semg[b]).wait()

        def process_compute(b):
            for j in range(CHUNK // L):
                sl = pl.ds(j * L, L)
                dst_v[b][sl] = meta_v[b][pl.ds(2 * CHUNK + j * L, L)]

            def group_body(gi, _):
                nv = norm_v[b][pl.ds(gi * L, L)]
                for lane in range(L):
                    ei = gi * L + lane
                    nb = jnp.full((L,), nv[lane], jnp.float32)
                    for j in range(nj):
                        sl = pl.ds(j * L, L)
                        rows_v[b][ei, sl] = rows_v[b][ei, sl] * nb
                return 0
            lax.fori_loop(0, CHUNK // L, group_body, 0)

        def scatter_start(b):
            pltpu.async_copy(rows_v[b], acc.at[dst_v[b]], sems[b], add=True)

        def scatter_wait(b):
            pltpu.make_async_copy(rows_v[b], acc.at[dst_v[b]],
                                  sems[b]).wait()

        def zstore(i, _):
            for j in range(nj):
                rows_v0[i, pl.ds(j * L, L)] = jnp.zeros((L,), jnp.float32)
            return 0
        lax.fori_loop(0, CHUNK, zstore, 0)
        for t in range(rcnt // CHUNK):
            pltpu.sync_copy(rows_v0,
                            acc.at[pl.ds(s * rstride + t * CHUNK, CHUNK)])
        plsc.subcore_barrier()

        meta_start(0, 0)
        meta_wait(0)
        idx_compute(0)
        gather_start(0)
        meta_start(1, 1)

        def half_step(k, cur, nxt, first=False):
            meta_wait(nxt)
            idx_compute(nxt)
            gather_wait(cur)
            if not first:
                scatter_wait(nxt)
            gather_start(nxt)
            process_compute(cur)
            meta_start(k + 2, cur)
            scatter_start(cur)

        half_step(0, 0, 1, first=True)
        half_step(1, 1, 0)

        def pair_body(t, _):
            half_step(2 * t, 0, 1)
            half_step(2 * t + 1, 1, 0)
            return 0
        lax.fori_loop(1, (nchunks - 1) // 2, pair_body, 0)

        meta_wait(1)
        gather_wait(0)
        scatter_wait(1)
        process_compute(0)
        scatter_start(0)
        scatter_wait(0)

        plsc.subcore_barrier()
        pltpu.sync_copy(acc.at[pl.ds(s * rstride, rcnt)],
                        out_hbm.at[c, pl.ds(s * rstride, rcnt)])

    return sc_kernel


def kernel(g, x, etypes, norm, weight, h_bias):
    n, din = x.shape
    r, _, dout = weight.shape
    e = g.shape[1]
    nch = e // CHUNK
    meta = jnp.concatenate(
        [g[0].reshape(nch, CHUNK), etypes.reshape(nch, CHUNK),
         g[1].reshape(nch, CHUNK)],
        axis=1).reshape(-1)
    h_flat = _rel_transforms(x, weight).reshape(r * n, dout)
    partial = _make_sc_scatter(n, dout, e)(meta, norm.reshape(-1), h_flat)
    return _combine(partial, h_bias)


# speedup vs baseline: 1.2569x; 1.0539x over previous
import functools

import jax
import jax.numpy as jnp
from jax import lax
from jax.experimental import pallas as pl
from jax.experimental.pallas import tpu as pltpu
from jax.experimental.pallas import tpu_sc as plsc

L = 16
NC = 2
NS = 16
CHUNK = 80
NB = 4


def _make_matmul_body(r):
    def _matmul_body(x_ref, w_ref, o_ref):
        for ri in range(r):
            o_ref[ri] = jnp.dot(x_ref[...], w_ref[ri],
                                preferred_element_type=jnp.float32)
    return _matmul_body


def _rel_transforms(x, weight):
    n, din = x.shape
    r, _, dout = weight.shape
    bn = 1000
    return pl.pallas_call(
        _make_matmul_body(r),
        grid=(n // bn,),
        in_specs=[
            pl.BlockSpec((bn, din), lambda ni: (ni, 0)),
            pl.BlockSpec((r, din, dout), lambda ni: (0, 0, 0)),
        ],
        out_specs=pl.BlockSpec((r, bn, dout), lambda ni: (0, ni, 0)),
        out_shape=jax.ShapeDtypeStruct((r, n, dout), jnp.float32),
    )(x, weight)


def _combine_body(p_ref, b_ref, o_ref):
    o_ref[...] = p_ref[0] + p_ref[1] + b_ref[...]


def _combine(partial, h_bias):
    nc, n, dout = partial.shape
    bn = 1000
    return pl.pallas_call(
        _combine_body,
        grid=(n // bn,),
        in_specs=[
            pl.BlockSpec((nc, bn, dout), lambda i: (0, i, 0)),
            pl.BlockSpec((1, dout), lambda i: (0, 0)),
        ],
        out_specs=pl.BlockSpec((bn, dout), lambda i: (i, 0)),
        out_shape=jax.ShapeDtypeStruct((n, dout), jnp.float32),
    )(partial, h_bias.reshape(1, dout))


def _make_sc_scatter(n, dout, e):
    nw = NC * NS
    ew = e // nw
    nchunks = ew // CHUNK
    rstride = 624
    rcnt = 640
    nj = dout // L
    mesh = plsc.VectorSubcoreMesh(core_axis_name="c", subcore_axis_name="s")
    mw = 3 * CHUNK
    gmax = nw * nchunks - 1

    @functools.partial(
        pl.kernel,
        out_type=jax.ShapeDtypeStruct((NC, n, dout), jnp.float32),
        mesh=mesh,
        scratch_types=(
            [pltpu.VMEM((mw,), jnp.int32) for _ in range(NB)]
            + [pltpu.VMEM((CHUNK,), jnp.int32) for _ in range(NB)]
            + [pltpu.VMEM((CHUNK,), jnp.int32) for _ in range(NB)]
            + [pltpu.VMEM((CHUNK,), jnp.float32) for _ in range(NB)]
            + [pltpu.VMEM((CHUNK, 128), jnp.float32) for _ in range(NB)]
            + [pltpu.VMEM_SHARED((n, dout), jnp.float32)]
            + [pltpu.SemaphoreType.DMA for _ in range(4 * NB)]
        ),
    )
    def sc_kernel(meta_hbm, norm_hbm, h_hbm, out_hbm, *sc):
        meta_v = sc[0:NB]
        idx_v = sc[NB:2 * NB]
        dst_v = sc[2 * NB:3 * NB]
        norm_v = sc[3 * NB:4 * NB]
        rows_v = sc[4 * NB:5 * NB]
        acc = sc[5 * NB]
        semm = sc[5 * NB + 1:5 * NB + 1 + NB]
        semn = sc[5 * NB + 1 + NB:5 * NB + 1 + 2 * NB]
        semg = sc[5 * NB + 1 + 2 * NB:5 * NB + 1 + 3 * NB]
        sems = sc[5 * NB + 1 + 3 * NB:5 * NB + 1 + 4 * NB]
        c = lax.axis_index("c")
        s = lax.axis_index("s")
        wid = c * NS + s

        def meta_start(k, b):
            g = jnp.minimum(wid * nchunks + k, gmax)
            pltpu.async_copy(
                meta_hbm.at[pl.ds(g * mw, mw)], meta_v[b], semm[b])
            pltpu.async_copy(
                norm_hbm.at[pl.ds(g * CHUNK, CHUNK)], norm_v[b], semn[b])

        def meta_wait(b):
            pltpu.make_async_copy(
                meta_hbm.at[pl.ds(0, mw)], meta_v[b], semm[b]).wait()
            pltpu.make_async_copy(
                norm_hbm.at[pl.ds(0, CHUNK)], norm_v[b], semn[b]).wait()

        def idx_compute(b):
            for j in range(CHUNK // L):
                sl = pl.ds(j * L, L)
                idx_v[b][sl] = (meta_v[b][pl.ds(CHUNK + j * L, L)] * n
                                + meta_v[b][sl])

        def gather_start(b):
            pltpu.async_copy(h_hbm.at[idx_v[b]], rows_v[b], semg[b])

        def gather_wait(b):
            pltpu.make_async_copy(
                h_hbm.at[idx_v[b]], rows_v[b], semg[b]).wait()

        def process_compute(b):
            for j in range(CHUNK // L):
                sl = pl.ds(j * L, L)
                dst_v[b][sl] = meta_v[b][pl.ds(2 * CHUNK + j * L, L)]

            def group_body(gi, _):
                nv = norm_v[b][pl.ds(gi * L, L)]
                for lane in range(L):
                    ei = gi * L + lane
                    nb = jnp.full((L,), nv[lane], jnp.float32)
                    for j in range(nj):
                        sl = pl.ds(j * L, L)
                        rows_v[b][ei, sl] = rows_v[b][ei, sl] * nb
                return 0
            lax.fori_loop(0, CHUNK // L, group_body, 0)

        def scatter_start(b):
            pltpu.async_copy(rows_v[b], acc.at[dst_v[b]], sems[b], add=True)

        def scatter_wait(b):
            pltpu.make_async_copy(rows_v[b], acc.at[dst_v[b]],
                                  sems[b]).wait()

        def zstore(i, _):
            for j in range(nj):
                rows_v[0][i, pl.ds(j * L, L)] = jnp.zeros((L,), jnp.float32)
            return 0
        lax.fori_loop(0, CHUNK, zstore, 0)
        for t in range(rcnt // CHUNK):
            pltpu.sync_copy(rows_v[0],
                            acc.at[pl.ds(s * rstride + t * CHUNK, CHUNK)])
        plsc.subcore_barrier()

        def step(k, b, first=False):
            b3 = (b + 3) % NB
            meta_wait(b3)
            idx_compute(b3)
            gather_wait(b)
            if not first:
                scatter_wait(b3)
            gather_start(b3)
            process_compute(b)
            meta_start(k + NB, b)
            scatter_start(b)

        for b in range(3):
            meta_start(b, b)
        for b in range(3):
            meta_wait(b)
            idx_compute(b)
            gather_start(b)
        meta_start(3, 3)

        step(0, 0, first=True)
        step(1, 1)
        step(2, 2)
        step(3, 3)

        def ring_body(t, _):
            k = 4 * t
            step(k, 0)
            step(k + 1, 1)
            step(k + 2, 2)
            step(k + 3, 3)
            return 0
        lax.fori_loop(1, (nchunks - 1) // NB, ring_body, 0)

        step(nchunks - 1, 0)
        meta_wait(0)
        gather_wait(1)
        gather_wait(2)
        gather_wait(3)
        scatter_wait(0)

        plsc.subcore_barrier()
        pltpu.sync_copy(acc.at[pl.ds(s * rstride, rcnt)],
                        out_hbm.at[c, pl.ds(s * rstride, rcnt)])

    return sc_kernel


def kernel(g, x, etypes, norm, weight, h_bias):
    n, din = x.shape
    r, _, dout = weight.shape
    e = g.shape[1]
    nch = e // CHUNK
    meta = jnp.concatenate(
        [g[0].reshape(nch, CHUNK), etypes.reshape(nch, CHUNK),
         g[1].reshape(nch, CHUNK)],
        axis=1).reshape(-1)
    h_flat = _rel_transforms(x, weight).reshape(r * n, dout)
    partial = _make_sc_scatter(n, dout, e)(meta, norm.reshape(-1), h_flat)
    return _combine(partial, h_bias)


# speedup vs baseline: 1.3428x; 1.0684x over previous
import functools

import jax
import jax.numpy as jnp
from jax import lax
from jax.experimental import pallas as pl
from jax.experimental.pallas import tpu as pltpu
from jax.experimental.pallas import tpu_sc as plsc

L = 16
NC = 2
NS = 16
CHUNK = 80
NB = 4


def _make_matmul_body(r):
    def _matmul_body(x_ref, w_ref, o_ref):
        for ri in range(r):
            o_ref[ri] = jnp.dot(x_ref[...], w_ref[ri],
                                preferred_element_type=jnp.float32)
    return _matmul_body


def _rel_transforms(x, weight):
    n, din = x.shape
    r, _, dout = weight.shape
    bn = 1000
    return pl.pallas_call(
        _make_matmul_body(r),
        grid=(n // bn,),
        in_specs=[
            pl.BlockSpec((bn, din), lambda ni: (ni, 0)),
            pl.BlockSpec((r, din, dout), lambda ni: (0, 0, 0)),
        ],
        out_specs=pl.BlockSpec((r, bn, dout), lambda ni: (0, ni, 0)),
        out_shape=jax.ShapeDtypeStruct((r, n, dout), jnp.float32),
    )(x, weight)


def _combine_body(p_ref, b_ref, o_ref):
    o_ref[...] = p_ref[0] + p_ref[1] + b_ref[...]


def _combine(partial, h_bias):
    nc, n, dout = partial.shape
    bn = 1000
    return pl.pallas_call(
        _combine_body,
        grid=(n // bn,),
        in_specs=[
            pl.BlockSpec((nc, bn, dout), lambda i: (0, i, 0)),
            pl.BlockSpec((1, dout), lambda i: (0, 0)),
        ],
        out_specs=pl.BlockSpec((bn, dout), lambda i: (i, 0)),
        out_shape=jax.ShapeDtypeStruct((n, dout), jnp.float32),
    )(partial, h_bias.reshape(1, dout))


def _make_sc_scatter(n, dout, e):
    nw = NC * NS
    ew = e // nw
    nchunks = ew // CHUNK
    rstride = 624
    rcnt = 640
    nj = dout // L
    mesh = plsc.VectorSubcoreMesh(core_axis_name="c", subcore_axis_name="s")
    mw = 3 * CHUNK
    gmax = nw * nchunks - 1

    @functools.partial(
        pl.kernel,
        out_type=jax.ShapeDtypeStruct((NC, n, dout), jnp.float32),
        mesh=mesh,
        scratch_types=(
            [pltpu.VMEM((CHUNK,), jnp.int32) for _ in range(3 * NB)]
            + [pltpu.VMEM((CHUNK,), jnp.int32) for _ in range(NB)]
            + [pltpu.VMEM((CHUNK,), jnp.int32) for _ in range(NB)]
            + [pltpu.VMEM((CHUNK,), jnp.float32) for _ in range(NB)]
            + [pltpu.VMEM((CHUNK, 128), jnp.float32) for _ in range(NB)]
            + [pltpu.VMEM_SHARED((n, dout), jnp.float32)]
            + [pltpu.SemaphoreType.DMA for _ in range(4 * NB)]
        ),
    )
    def sc_kernel(src_hbm, et_hbm, dstm_hbm, norm_hbm, h_hbm, out_hbm, *sc):
        src_v = sc[0:NB]
        et_v = sc[NB:2 * NB]
        dstm_v = sc[2 * NB:3 * NB]
        idx_v = sc[3 * NB:4 * NB]
        dst_v = sc[4 * NB:5 * NB]
        norm_v = sc[5 * NB:6 * NB]
        rows_v = sc[6 * NB:7 * NB]
        acc = sc[7 * NB]
        semm = sc[7 * NB + 1:7 * NB + 1 + NB]
        semn = sc[7 * NB + 1 + NB:7 * NB + 1 + 2 * NB]
        semg = sc[7 * NB + 1 + 2 * NB:7 * NB + 1 + 3 * NB]
        sems = sc[7 * NB + 1 + 3 * NB:7 * NB + 1 + 4 * NB]
        c = lax.axis_index("c")
        s = lax.axis_index("s")
        wid = c * NS + s

        def meta_start(k, b):
            g = jnp.minimum(wid * nchunks + k, gmax)
            base = pl.ds(g * CHUNK, CHUNK)
            pltpu.async_copy(src_hbm.at[base], src_v[b], semm[b])
            pltpu.async_copy(et_hbm.at[base], et_v[b], semm[b])
            pltpu.async_copy(dstm_hbm.at[base], dstm_v[b], semm[b])
            pltpu.async_copy(norm_hbm.at[base], norm_v[b], semn[b])

        def meta_wait(b):
            base = pl.ds(0, CHUNK)
            pltpu.make_async_copy(src_hbm.at[base], src_v[b], semm[b]).wait()
            pltpu.make_async_copy(et_hbm.at[base], et_v[b], semm[b]).wait()
            pltpu.make_async_copy(dstm_hbm.at[base], dstm_v[b], semm[b]).wait()
            pltpu.make_async_copy(norm_hbm.at[base], norm_v[b], semn[b]).wait()

        def idx_compute(b):
            for j in range(CHUNK // L):
                sl = pl.ds(j * L, L)
                idx_v[b][sl] = et_v[b][sl] * n + src_v[b][sl]

        def gather_start(b):
            pltpu.async_copy(h_hbm.at[idx_v[b]], rows_v[b], semg[b])

        def gather_wait(b):
            pltpu.make_async_copy(
                h_hbm.at[idx_v[b]], rows_v[b], semg[b]).wait()

        def process_compute(b):
            for j in range(CHUNK // L):
                sl = pl.ds(j * L, L)
                dst_v[b][sl] = dstm_v[b][sl]

            def group_body(gi, _):
                nv = norm_v[b][pl.ds(gi * L, L)]
                for lane in range(L):
                    ei = gi * L + lane
                    nb = jnp.full((L,), nv[lane], jnp.float32)
                    for j in range(nj):
                        sl = pl.ds(j * L, L)
                        rows_v[b][ei, sl] = rows_v[b][ei, sl] * nb
                return 0
            lax.fori_loop(0, CHUNK // L, group_body, 0)

        def scatter_start(b):
            pltpu.async_copy(rows_v[b], acc.at[dst_v[b]], sems[b], add=True)

        def scatter_wait(b):
            pltpu.make_async_copy(rows_v[b], acc.at[dst_v[b]],
                                  sems[b]).wait()

        def zstore(i, _):
            for j in range(nj):
                rows_v[0][i, pl.ds(j * L, L)] = jnp.zeros((L,), jnp.float32)
            return 0
        lax.fori_loop(0, CHUNK, zstore, 0)
        for t in range(rcnt // CHUNK):
            pltpu.sync_copy(rows_v[0],
                            acc.at[pl.ds(s * rstride + t * CHUNK, CHUNK)])
        plsc.subcore_barrier()

        def step(k, b, first=False):
            b3 = (b + 3) % NB
            meta_wait(b3)
            idx_compute(b3)
            gather_wait(b)
            if not first:
                scatter_wait(b3)
            gather_start(b3)
            process_compute(b)
            meta_start(k + NB, b)
            scatter_start(b)

        for b in range(3):
            meta_start(b, b)
        for b in range(3):
            meta_wait(b)
            idx_compute(b)
            gather_start(b)
        meta_start(3, 3)

        step(0, 0, first=True)
        step(1, 1)
        step(2, 2)
        step(3, 3)

        def ring_body(t, _):
            k = 4 * t
            step(k, 0)
            step(k + 1, 1)
            step(k + 2, 2)
            step(k + 3, 3)
            return 0
        lax.fori_loop(1, (nchunks - 1) // NB, ring_body, 0)

        step(nchunks - 1, 0)
        meta_wait(0)
        gather_wait(1)
        gather_wait(2)
        gather_wait(3)
        scatter_wait(0)

        plsc.subcore_barrier()
        pltpu.sync_copy(acc.at[pl.ds(s * rstride, rcnt)],
                        out_hbm.at[c, pl.ds(s * rstride, rcnt)])

    return sc_kernel


def kernel(g, x, etypes, norm, weight, h_bias):
    n, din = x.shape
    r, _, dout = weight.shape
    e = g.shape[1]
    h_flat = _rel_transforms(x, weight).reshape(r * n, dout)
    partial = _make_sc_scatter(n, dout, e)(
        g[0], etypes, g[1], norm.reshape(-1), h_flat)
    return _combine(partial, h_bias)


# speedup vs baseline: 1.3516x; 1.0066x over previous
import functools

import jax
import jax.numpy as jnp
from jax import lax
from jax.experimental import pallas as pl
from jax.experimental.pallas import tpu as pltpu
from jax.experimental.pallas import tpu_sc as plsc

L = 16
NC = 2
NS = 16
CHUNK = 80
NB = 4


def _make_matmul_body(r):
    def _matmul_body(x_ref, w_ref, o_ref):
        for ri in range(r):
            o_ref[ri] = jnp.dot(x_ref[...], w_ref[ri],
                                preferred_element_type=jnp.float32)
    return _matmul_body


def _rel_transforms(x, weight):
    n, din = x.shape
    r, _, dout = weight.shape
    bn = 1000
    return pl.pallas_call(
        _make_matmul_body(r),
        grid=(n // bn,),
        in_specs=[
            pl.BlockSpec((bn, din), lambda ni: (ni, 0)),
            pl.BlockSpec((r, din, dout), lambda ni: (0, 0, 0)),
        ],
        out_specs=pl.BlockSpec((r, bn, dout), lambda ni: (0, ni, 0)),
        out_shape=jax.ShapeDtypeStruct((r, n, dout), jnp.float32),
    )(x, weight)


def _combine_body(p_ref, b_ref, o_ref):
    o_ref[...] = p_ref[0] + p_ref[1] + b_ref[...]


def _combine(partial, h_bias):
    nc, n, dout = partial.shape
    bn = 1000
    return pl.pallas_call(
        _combine_body,
        grid=(n // bn,),
        in_specs=[
            pl.BlockSpec((nc, bn, dout), lambda i: (0, i, 0)),
            pl.BlockSpec((1, dout), lambda i: (0, 0)),
        ],
        out_specs=pl.BlockSpec((bn, dout), lambda i: (i, 0)),
        out_shape=jax.ShapeDtypeStruct((n, dout), jnp.float32),
    )(partial, h_bias.reshape(1, dout))


def _make_sc_scatter(n, dout, e):
    nw = NC * NS
    ew = e // nw
    nchunks = ew // CHUNK
    rstride = 624
    rcnt = 640
    nj = dout // L
    mesh = plsc.VectorSubcoreMesh(core_axis_name="c", subcore_axis_name="s")
    mw = 3 * CHUNK
    gmax = nw * nchunks - 1

    @functools.partial(
        pl.kernel,
        out_type=jax.ShapeDtypeStruct((NC, n, dout), jnp.float32),
        mesh=mesh,
        scratch_types=(
            [pltpu.VMEM((CHUNK,), jnp.int32) for _ in range(3 * NB)]
            + [pltpu.VMEM((CHUNK,), jnp.int32) for _ in range(NB)]
            + [pltpu.VMEM((CHUNK,), jnp.int32) for _ in range(NB)]
            + [pltpu.VMEM((CHUNK,), jnp.float32) for _ in range(NB)]
            + [pltpu.VMEM((CHUNK, 128), jnp.float32) for _ in range(NB)]
            + [pltpu.VMEM_SHARED((n, dout), jnp.float32)]
            + [pltpu.SemaphoreType.DMA for _ in range(5 * NB)]
        ),
    )
    def sc_kernel(src_hbm, et_hbm, dstm_hbm, norm_hbm, h_hbm, out_hbm, *sc):
        src_v = sc[0:NB]
        et_v = sc[NB:2 * NB]
        dstm_v = sc[2 * NB:3 * NB]
        idx_v = sc[3 * NB:4 * NB]
        dst_v = sc[4 * NB:5 * NB]
        norm_v = sc[5 * NB:6 * NB]
        rows_v = sc[6 * NB:7 * NB]
        acc = sc[7 * NB]
        semm = sc[7 * NB + 1:7 * NB + 1 + NB]
        semn = sc[7 * NB + 1 + NB:7 * NB + 1 + 2 * NB]
        semg = sc[7 * NB + 1 + 2 * NB:7 * NB + 1 + 3 * NB]
        sems = sc[7 * NB + 1 + 3 * NB:7 * NB + 1 + 4 * NB]
        semg2 = sc[7 * NB + 1 + 4 * NB:7 * NB + 1 + 5 * NB]
        c = lax.axis_index("c")
        s = lax.axis_index("s")
        wid = c * NS + s

        def meta_start(k, b):
            g = jnp.minimum(wid * nchunks + k, gmax)
            base = pl.ds(g * CHUNK, CHUNK)
            pltpu.async_copy(src_hbm.at[base], src_v[b], semm[b])
            pltpu.async_copy(et_hbm.at[base], et_v[b], semm[b])
            pltpu.async_copy(dstm_hbm.at[base], dstm_v[b], semm[b])
            pltpu.async_copy(norm_hbm.at[base], norm_v[b], semn[b])

        def meta_wait(b):
            base = pl.ds(0, CHUNK)
            pltpu.make_async_copy(src_hbm.at[base], src_v[b], semm[b]).wait()
            pltpu.make_async_copy(et_hbm.at[base], et_v[b], semm[b]).wait()
            pltpu.make_async_copy(dstm_hbm.at[base], dstm_v[b], semm[b]).wait()
            pltpu.make_async_copy(norm_hbm.at[base], norm_v[b], semn[b]).wait()

        def idx_compute(b):
            for j in range(CHUNK // L):
                sl = pl.ds(j * L, L)
                idx_v[b][sl] = et_v[b][sl] * n + src_v[b][sl]

        H = CHUNK // 2

        def gather_start(b):
            pltpu.async_copy(h_hbm.at[idx_v[b].at[pl.ds(0, H)]],
                             rows_v[b].at[pl.ds(0, H)], semg[b])
            pltpu.async_copy(h_hbm.at[idx_v[b].at[pl.ds(H, H)]],
                             rows_v[b].at[pl.ds(H, H)], semg2[b])

        def gather_wait(b):
            pltpu.make_async_copy(h_hbm.at[idx_v[b].at[pl.ds(0, H)]],
                                  rows_v[b].at[pl.ds(0, H)], semg[b]).wait()
            pltpu.make_async_copy(h_hbm.at[idx_v[b].at[pl.ds(H, H)]],
                                  rows_v[b].at[pl.ds(H, H)], semg2[b]).wait()

        def process_compute(b):
            for j in range(CHUNK // L):
                sl = pl.ds(j * L, L)
                dst_v[b][sl] = dstm_v[b][sl]

            def group_body(gi, _):
                nv = norm_v[b][pl.ds(gi * L, L)]
                for lane in range(L):
                    ei = gi * L + lane
                    nb = jnp.full((L,), nv[lane], jnp.float32)
                    for j in range(nj):
                        sl = pl.ds(j * L, L)
                        rows_v[b][ei, sl] = rows_v[b][ei, sl] * nb
                return 0
            lax.fori_loop(0, CHUNK // L, group_body, 0)

        def scatter_start(b):
            pltpu.async_copy(rows_v[b], acc.at[dst_v[b]], sems[b], add=True)

        def scatter_wait(b):
            pltpu.make_async_copy(rows_v[b], acc.at[dst_v[b]],
                                  sems[b]).wait()

        def zstore(i, _):
            for j in range(nj):
                rows_v[0][i, pl.ds(j * L, L)] = jnp.zeros((L,), jnp.float32)
            return 0
        lax.fori_loop(0, CHUNK, zstore, 0)
        for t in range(rcnt // CHUNK):
            pltpu.sync_copy(rows_v[0],
                            acc.at[pl.ds(s * rstride + t * CHUNK, CHUNK)])
        plsc.subcore_barrier()

        def step(k, b, first=False):
            b3 = (b + 3) % NB
            meta_wait(b3)
            idx_compute(b3)
            gather_wait(b)
            if not first:
                scatter_wait(b3)
            gather_start(b3)
            process_compute(b)
            meta_start(k + NB, b)
            scatter_start(b)

        for b in range(3):
            meta_start(b, b)
        for b in range(3):
            meta_wait(b)
            idx_compute(b)
            gather_start(b)
        meta_start(3, 3)

        step(0, 0, first=True)
        step(1, 1)
        step(2, 2)
        step(3, 3)

        def ring_body(t, _):
            k = 4 * t
            step(k, 0)
            step(k + 1, 1)
            step(k + 2, 2)
            step(k + 3, 3)
            return 0
        lax.fori_loop(1, (nchunks - 1) // NB, ring_body, 0)

        step(nchunks - 1, 0)
        meta_wait(0)
        gather_wait(1)
        gather_wait(2)
        gather_wait(3)
        scatter_wait(0)

        plsc.subcore_barrier()
        pltpu.sync_copy(acc.at[pl.ds(s * rstride, rcnt)],
                        out_hbm.at[c, pl.ds(s * rstride, rcnt)])

    return sc_kernel


def kernel(g, x, etypes, norm, weight, h_bias):
    n, din = x.shape
    r, _, dout = weight.shape
    e = g.shape[1]
    h_flat = _rel_transforms(x, weight).reshape(r * n, dout)
    partial = _make_sc_scatter(n, dout, e)(
        g[0], etypes, g[1], norm.reshape(-1), h_flat)
    return _combine(partial, h_bias)


# speedup vs baseline: 1.3595x; 1.0058x over previous
import functools

import jax
import jax.numpy as jnp
from jax import lax
from jax.experimental import pallas as pl
from jax.experimental.pallas import tpu as pltpu
from jax.experimental.pallas import tpu_sc as plsc

L = 16
NC = 2
NS = 16
CHUNK = 80
NB = 4


def _make_matmul_body(r):
    def _matmul_body(x_ref, w_ref, o_ref):
        for ri in range(r):
            o_ref[ri] = jnp.dot(x_ref[...], w_ref[ri],
                                preferred_element_type=jnp.float32)
    return _matmul_body


def _rel_transforms(x, weight):
    n, din = x.shape
    r, _, dout = weight.shape
    bn = 1000
    return pl.pallas_call(
        _make_matmul_body(r),
        grid=(n // bn,),
        in_specs=[
            pl.BlockSpec((bn, din), lambda ni: (ni, 0)),
            pl.BlockSpec((r, din, dout), lambda ni: (0, 0, 0)),
        ],
        out_specs=pl.BlockSpec((r, bn, dout), lambda ni: (0, ni, 0)),
        out_shape=jax.ShapeDtypeStruct((r, n, dout), jnp.float32),
    )(x, weight)


def _combine_body(p_ref, b_ref, o_ref):
    o_ref[...] = p_ref[0] + p_ref[1] + b_ref[...]


def _combine(partial, h_bias):
    nc, n, dout = partial.shape
    bn = 1000
    return pl.pallas_call(
        _combine_body,
        grid=(n // bn,),
        in_specs=[
            pl.BlockSpec((nc, bn, dout), lambda i: (0, i, 0)),
            pl.BlockSpec((1, dout), lambda i: (0, 0)),
        ],
        out_specs=pl.BlockSpec((bn, dout), lambda i: (i, 0)),
        out_shape=jax.ShapeDtypeStruct((n, dout), jnp.float32),
    )(partial, h_bias.reshape(1, dout))


def _make_sc_scatter(n, dout, e):
    nw = NC * NS
    ew = e // nw
    nchunks = ew // CHUNK
    rstride = 624
    rcnt = 640
    nj = dout // L
    mesh = plsc.VectorSubcoreMesh(core_axis_name="c", subcore_axis_name="s")
    mw = 3 * CHUNK
    gmax = nw * nchunks - 1

    @functools.partial(
        pl.kernel,
        out_type=jax.ShapeDtypeStruct((NC, n, dout), jnp.float32),
        mesh=mesh,
        scratch_types=(
            [pltpu.VMEM((CHUNK,), jnp.int32) for _ in range(3 * NB)]
            + [pltpu.VMEM((CHUNK,), jnp.int32) for _ in range(NB)]
            + [pltpu.VMEM((CHUNK,), jnp.int32) for _ in range(NB)]
            + [pltpu.VMEM((CHUNK,), jnp.float32) for _ in range(NB)]
            + [pltpu.VMEM((CHUNK, 128), jnp.float32) for _ in range(NB)]
            + [pltpu.VMEM_SHARED((n, dout), jnp.float32)]
            + [pltpu.SemaphoreType.DMA for _ in range(5 * NB)]
        ),
    )
    def sc_kernel(src_hbm, et_hbm, dstm_hbm, norm_hbm, h_hbm, out_hbm, *sc):
        src_v = sc[0:NB]
        et_v = sc[NB:2 * NB]
        dstm_v = sc[2 * NB:3 * NB]
        idx_v = sc[3 * NB:4 * NB]
        dst_v = sc[4 * NB:5 * NB]
        norm_v = sc[5 * NB:6 * NB]
        rows_v = sc[6 * NB:7 * NB]
        acc = sc[7 * NB]
        semm = sc[7 * NB + 1:7 * NB + 1 + NB]
        semn = sc[7 * NB + 1 + NB:7 * NB + 1 + 2 * NB]
        semg = sc[7 * NB + 1 + 2 * NB:7 * NB + 1 + 3 * NB]
        sems = sc[7 * NB + 1 + 3 * NB:7 * NB + 1 + 4 * NB]
        semg2 = sc[7 * NB + 1 + 4 * NB:7 * NB + 1 + 5 * NB]
        c = lax.axis_index("c")
        s = lax.axis_index("s")
        wid = c * NS + s

        def meta_start(k, b):
            g = jnp.minimum(wid * nchunks + k, gmax)
            base = pl.ds(g * CHUNK, CHUNK)
            pltpu.async_copy(src_hbm.at[base], src_v[b], semm[b])
            pltpu.async_copy(et_hbm.at[base], et_v[b], semm[b])
            pltpu.async_copy(dstm_hbm.at[base], dstm_v[b], semm[b])
            pltpu.async_copy(norm_hbm.at[base], norm_v[b], semn[b])

        def meta_wait(b):
            base = pl.ds(0, CHUNK)
            pltpu.make_async_copy(src_hbm.at[base], src_v[b], semm[b]).wait()
            pltpu.make_async_copy(et_hbm.at[base], et_v[b], semm[b]).wait()
            pltpu.make_async_copy(dstm_hbm.at[base], dstm_v[b], semm[b]).wait()
            pltpu.make_async_copy(norm_hbm.at[base], norm_v[b], semn[b]).wait()

        def idx_compute(b):
            for j in range(CHUNK // L):
                sl = pl.ds(j * L, L)
                idx_v[b][sl] = et_v[b][sl] * n + src_v[b][sl]

        H = CHUNK // 2

        def gather_start(b):
            pltpu.async_copy(h_hbm.at[idx_v[b].at[pl.ds(0, H)]],
                             rows_v[b].at[pl.ds(0, H)], semg[b])
            pltpu.async_copy(h_hbm.at[idx_v[b].at[pl.ds(H, H)]],
                             rows_v[b].at[pl.ds(H, H)], semg2[b])

        def gather_wait(b):
            pltpu.make_async_copy(h_hbm.at[idx_v[b].at[pl.ds(0, H)]],
                                  rows_v[b].at[pl.ds(0, H)], semg[b]).wait()
            pltpu.make_async_copy(h_hbm.at[idx_v[b].at[pl.ds(H, H)]],
                                  rows_v[b].at[pl.ds(H, H)], semg2[b]).wait()

        def process_compute(b):
            for j in range(CHUNK // L):
                sl = pl.ds(j * L, L)
                dst_v[b][sl] = dstm_v[b][sl]

            def group_body(gi, _):
                nv = norm_v[b][pl.ds(gi * L, L)]
                for lane in range(L):
                    ei = gi * L + lane
                    nb = jnp.full((L,), nv[lane], jnp.float32)
                    for j in range(nj):
                        sl = pl.ds(j * L, L)
                        rows_v[b][ei, sl] = rows_v[b][ei, sl] * nb
                return 0
            lax.fori_loop(0, CHUNK // L, group_body, 0)

        def scatter_start(b):
            pltpu.async_copy(rows_v[b], acc.at[dst_v[b]], sems[b], add=True)

        def scatter_wait(b):
            pltpu.make_async_copy(rows_v[b], acc.at[dst_v[b]],
                                  sems[b]).wait()

        for b in range(3):
            meta_start(b, b)

        def zstore(i, _):
            for j in range(nj):
                rows_v[3][i, pl.ds(j * L, L)] = jnp.zeros((L,), jnp.float32)
            return 0
        lax.fori_loop(0, CHUNK, zstore, 0)

        for b in range(3):
            meta_wait(b)
            idx_compute(b)
            gather_start(b)
        meta_start(3, 3)

        for t in range(rcnt // CHUNK):
            pltpu.sync_copy(rows_v[3],
                            acc.at[pl.ds(s * rstride + t * CHUNK, CHUNK)])
        plsc.subcore_barrier()

        def step(k, b, first=False):
            b3 = (b + 3) % NB
            meta_wait(b3)
            idx_compute(b3)
            gather_wait(b)
            if not first:
                scatter_wait(b3)
            gather_start(b3)
            process_compute(b)
            meta_start(k + NB, b)
            scatter_start(b)

        step(0, 0, first=True)
        step(1, 1)
        step(2, 2)
        step(3, 3)

        def ring_body(t, _):
            k = 4 * t
            step(k, 0)
            step(k + 1, 1)
            step(k + 2, 2)
            step(k + 3, 3)
            return 0
        lax.fori_loop(1, (nchunks - 1) // NB, ring_body, 0)

        step(nchunks - 1, 0)
        meta_wait(0)
        gather_wait(1)
        gather_wait(2)
        gather_wait(3)
        scatter_wait(0)

        plsc.subcore_barrier()
        pltpu.sync_copy(acc.at[pl.ds(s * rstride, rcnt)],
                        out_hbm.at[c, pl.ds(s * rstride, rcnt)])

    return sc_kernel


def kernel(g, x, etypes, norm, weight, h_bias):
    n, din = x.shape
    r, _, dout = weight.shape
    e = g.shape[1]
    h_flat = _rel_transforms(x, weight).reshape(r * n, dout)
    partial = _make_sc_scatter(n, dout, e)(
        g[0], etypes, g[1], norm.reshape(-1), h_flat)
    return _combine(partial, h_bias)
